# baseline (speedup 1.0000x reference)
import functools

import jax
import jax.numpy as jnp
from jax import lax
from jax.experimental import pallas as pl
from jax.experimental.pallas import tpu as pltpu
from jax.experimental.pallas import tpu_sc as plsc

BATCH = 16384
D = 64
ROWS = 100000
NC = 2
NS = 16
NW = NC * NS
BPW = BATCH // NW
CHUNK = 128
NCHUNK = BPW // CHUNK
MBLKC = 8192


def _merge_pair(A, B):
    def body(a_r, b_r, t_r):
        i0 = lax.broadcasted_iota(jnp.int32, (D, D), 0)
        i1 = lax.broadcasted_iota(jnp.int32, (D, D), 1)
        eye = (i0 == i1).astype(jnp.float32)
        dn = (((0,), (0,)), ((), ()))

        def tr(x):
            return lax.dot_general(x, eye, dn,
                                   preferred_element_type=jnp.float32)

        t_r[:, 0:D] = tr(a_r[...])
        t_r[:, D:2 * D] = tr(b_r[...])

    spec_in = pl.BlockSpec((D, MBLKC), lambda i: (0, i))
    spec_out = pl.BlockSpec((MBLKC, 2 * D), lambda i: (i, 0))
    return pl.pallas_call(
        body,
        grid=(pl.cdiv(ROWS, MBLKC),),
        in_specs=[spec_in] * 2,
        out_specs=spec_out,
        out_shape=jax.ShapeDtypeStruct((ROWS, 2 * D), jnp.float32),
    )(A.T, B.T)


def _sc_gather_side(idx, T):
    mesh = plsc.VectorSubcoreMesh(
        core_axis_name="c", subcore_axis_name="s",
        num_cores=NC, num_subcores=NS)

    @functools.partial(
        pl.kernel,
        out_type=jax.ShapeDtypeStruct((BATCH, 2 * D), jnp.float32),
        mesh=mesh,
        scratch_types=(
            pltpu.VMEM((BPW,), jnp.int32),
            pltpu.VMEM((BPW, 2 * D), jnp.float32),
            pltpu.SemaphoreType.DMA,
        ),
    )
    def k(idx_h, t_h, g_h, idx_v, buf, sem):
        wid = lax.axis_index("s") * NC + lax.axis_index("c")
        base = wid * BPW
        pltpu.sync_copy(idx_h.at[pl.ds(base, BPW)], idx_v)
        cps = [
            pltpu.async_copy(
                t_h.at[idx_v.at[pl.ds(j * CHUNK, CHUNK)]],
                buf.at[pl.ds(j * CHUNK, CHUNK)], sem)
            for j in range(NCHUNK)
        ]
        for cp in cps:
            cp.wait()
        pltpu.sync_copy(buf, g_h.at[pl.ds(base, BPW)])

    return k(idx, T)


def _tc_mlp(gu, gi, W1u, W1i, b1, W2, b2, W3, b3, Wph, Wpm, bp):
    BT = 2048

    def body(gu_r, gi_r, w1u_r, w1i_r, b1_r, w2_r, b2_r, w3_r, b3_r,
             wph_r, wpm_r, bp_r, out_r):
        f32 = jnp.float32
        gu_v = gu_r[...]
        gi_v = gi_r[...]
        h = jnp.maximum(
            jnp.dot(gu_v, w1u_r[...], preferred_element_type=f32)
            + jnp.dot(gi_v, w1i_r[...], preferred_element_type=f32)
            + b1_r[...], 0.0)
        h = jnp.maximum(jnp.dot(h, w2_r[...], preferred_element_type=f32)
                        + b2_r[...], 0.0)
        h = jnp.maximum(jnp.dot(h, w3_r[...], preferred_element_type=f32)
                        + b3_r[...], 0.0)
        pred = (jnp.sum(h * wph_r[...], axis=1)
                + jnp.sum(gu_v * gi_v * wpm_r[...], axis=1)
                + bp_r[0, 0])
        out_r[...] = jax.nn.sigmoid(pred)

    def full(shape):
        return pl.BlockSpec(shape, lambda i: tuple(0 for _ in shape))

    return pl.pallas_call(
        body,
        grid=(BATCH // BT,),
        in_specs=[
            pl.BlockSpec((BT, 2 * D), lambda i: (i, 0)),
            pl.BlockSpec((BT, 2 * D), lambda i: (i, 0)),
            full((2 * D, 64)), full((2 * D, 64)), full((1, 64)),
            full((64, 32)), full((1, 32)),
            full((32, 16)), full((1, 16)),
            full((1, 16)), full((1, 2 * D)), full((1, 1)),
        ],
        out_specs=pl.BlockSpec((BT,), lambda i: (i,)),
        out_shape=jax.ShapeDtypeStruct((BATCH,), jnp.float32),
    )(gu, gi, W1u, W1i, b1, W2, b2, W3, b3, Wph, Wpm, bp)


def kernel(user, item, U_mf, I_mf, U_mlp, I_mlp,
           W1, b1, W2, b2, W3, b3, Wp, bp):
    Tu = _merge_pair(U_mlp, U_mf)
    gu = _sc_gather_side(user, Tu)
    Ti = _merge_pair(I_mlp, I_mf)
    gi = _sc_gather_side(item, Ti)
    zeros = jnp.zeros((D, 64), jnp.float32)
    W1u = jnp.concatenate([W1[:D], zeros], axis=0)
    W1i = jnp.concatenate([zeros, W1[D:]], axis=0)
    Wph = Wp[:16, 0][None, :]
    Wpm = jnp.concatenate(
        [jnp.zeros((1, D), jnp.float32), Wp[16:, 0][None, :]], axis=1)
    return _tc_mlp(
        gu, gi, W1u, W1i, b1[None, :],
        W2, b2[None, :], W3, b3[None, :],
        Wph, Wpm, bp.reshape(1, 1))


# speedup vs baseline: 1.2921x; 1.2921x over previous
import functools

import jax
import jax.numpy as jnp
from jax import lax
from jax.experimental import pallas as pl
from jax.experimental.pallas import tpu as pltpu
from jax.experimental.pallas import tpu_sc as plsc

BATCH = 16384
D = 64
ROWS = 100000
NC = 2
NS = 16
NW = NC * NS
BPW = BATCH // NW
CHUNK = 128
NCHUNK = BPW // CHUNK
MBLKC = 8192


def _merge_pair(A, B):
    def body(a_r, b_r, t_r):
        i0 = lax.broadcasted_iota(jnp.int32, (2 * D, 2 * D), 0)
        i1 = lax.broadcasted_iota(jnp.int32, (2 * D, 2 * D), 1)
        eye = (i0 == i1).astype(jnp.float32)
        x = jnp.concatenate([a_r[...], b_r[...]], axis=0)
        t_r[...] = lax.dot_general(
            x, eye, (((0,), (0,)), ((), ())),
            preferred_element_type=jnp.float32)

    spec_in = pl.BlockSpec((D, MBLKC), lambda i: (0, i))
    spec_out = pl.BlockSpec((MBLKC, 2 * D), lambda i: (i, 0))
    return pl.pallas_call(
        body,
        grid=(pl.cdiv(ROWS, MBLKC),),
        in_specs=[spec_in] * 2,
        out_specs=spec_out,
        out_shape=jax.ShapeDtypeStruct((ROWS, 2 * D), jnp.float32),
    )(A.T, B.T)


def _sc_gather_side(idx, T):
    mesh = plsc.VectorSubcoreMesh(
        core_axis_name="c", subcore_axis_name="s",
        num_cores=NC, num_subcores=NS)

    @functools.partial(
        pl.kernel,
        out_type=jax.ShapeDtypeStruct((BATCH, 2 * D), jnp.float32),
        mesh=mesh,
        scratch_types=(
            pltpu.VMEM((BPW,), jnp.int32),
            pltpu.VMEM((BPW, 2 * D), jnp.float32),
            pltpu.SemaphoreType.DMA,
        ),
    )
    def k(idx_h, t_h, g_h, idx_v, buf, sem):
        wid = lax.axis_index("s") * NC + lax.axis_index("c")
        base = wid * BPW
        pltpu.sync_copy(idx_h.at[pl.ds(base, BPW)], idx_v)
        cps = [
            pltpu.async_copy(
                t_h.at[idx_v.at[pl.ds(j * CHUNK, CHUNK)]],
                buf.at[pl.ds(j * CHUNK, CHUNK)], sem)
            for j in range(NCHUNK)
        ]
        for cp in cps:
            cp.wait()
        pltpu.sync_copy(buf, g_h.at[pl.ds(base, BPW)])

    return k(idx, T)


def _tc_mlp(gu, gi, W1u, W1i, b1, W2, b2, W3, b3, Wph, Wpm, bp):
    BT = 2048

    def body(gu_r, gi_r, w1u_r, w1i_r, b1_r, w2_r, b2_r, w3_r, b3_r,
             wph_r, wpm_r, bp_r, out_r):
        f32 = jnp.float32
        gu_v = gu_r[...]
        gi_v = gi_r[...]
        h = jnp.maximum(
            jnp.dot(gu_v, w1u_r[...], preferred_element_type=f32)
            + jnp.dot(gi_v, w1i_r[...], preferred_element_type=f32)
            + b1_r[...], 0.0)
        h = jnp.maximum(jnp.dot(h, w2_r[...], preferred_element_type=f32)
                        + b2_r[...], 0.0)
        h = jnp.maximum(jnp.dot(h, w3_r[...], preferred_element_type=f32)
                        + b3_r[...], 0.0)
        pred = (jnp.dot(h, wph_r[...], preferred_element_type=f32)
                + jnp.dot(gu_v * gi_v, wpm_r[...], preferred_element_type=f32)
                + bp_r[0, 0])
        out_r[...] = jax.nn.sigmoid(pred)

    def full(shape):
        return pl.BlockSpec(shape, lambda i: tuple(0 for _ in shape))

    return pl.pallas_call(
        body,
        grid=(BATCH // BT,),
        in_specs=[
            pl.BlockSpec((BT, 2 * D), lambda i: (i, 0)),
            pl.BlockSpec((BT, 2 * D), lambda i: (i, 0)),
            full((2 * D, 64)), full((2 * D, 64)), full((1, 64)),
            full((64, 32)), full((1, 32)),
            full((32, 16)), full((1, 16)),
            full((16, 1)), full((2 * D, 1)), full((1, 1)),
        ],
        out_specs=pl.BlockSpec((BT, 1), lambda i: (i, 0)),
        out_shape=jax.ShapeDtypeStruct((BATCH, 1), jnp.float32),
    )(gu, gi, W1u, W1i, b1, W2, b2, W3, b3, Wph, Wpm, bp)


def kernel(user, item, U_mf, I_mf, U_mlp, I_mlp,
           W1, b1, W2, b2, W3, b3, Wp, bp):
    Tu = _merge_pair(U_mlp, U_mf)
    gu = _sc_gather_side(user, Tu)
    Ti = _merge_pair(I_mlp, I_mf)
    gi = _sc_gather_side(item, Ti)
    zeros = jnp.zeros((D, 64), jnp.float32)
    W1u = jnp.concatenate([W1[:D], zeros], axis=0)
    W1i = jnp.concatenate([zeros, W1[D:]], axis=0)
    Wph = Wp[:16]
    Wpm = jnp.concatenate(
        [jnp.zeros((D, 1), jnp.float32), Wp[16:]], axis=0)
    pred = _tc_mlp(
        gu, gi, W1u, W1i, b1[None, :],
        W2, b2[None, :], W3, b3[None, :],
        Wph, Wpm, bp.reshape(1, 1))
    return pred[:, 0]


# speedup vs baseline: 1.3232x; 1.0241x over previous
import functools

import jax
import jax.numpy as jnp
from jax import lax
from jax.experimental import pallas as pl
from jax.experimental.pallas import tpu as pltpu
from jax.experimental.pallas import tpu_sc as plsc

BATCH = 16384
D = 64
ROWS = 100000
NC = 2
NS = 16
NW = NC * NS
BPW = BATCH // NW
CHUNK = 128
NCHUNK = BPW // CHUNK
MBLKC = 16384


def _merge_pair(A, B):
    def body(a_r, b_r, t_r):
        i0 = lax.broadcasted_iota(jnp.int32, (2 * D, 2 * D), 0)
        i1 = lax.broadcasted_iota(jnp.int32, (2 * D, 2 * D), 1)
        eye = (i0 == i1).astype(jnp.float32)
        x = jnp.concatenate([a_r[...], b_r[...]], axis=0)
        t_r[...] = lax.dot_general(
            x, eye, (((0,), (0,)), ((), ())),
            preferred_element_type=jnp.float32)

    spec_in = pl.BlockSpec((D, MBLKC), lambda i: (0, i))
    spec_out = pl.BlockSpec((MBLKC, 2 * D), lambda i: (i, 0))
    return pl.pallas_call(
        body,
        grid=(pl.cdiv(ROWS, MBLKC),),
        in_specs=[spec_in] * 2,
        out_specs=spec_out,
        out_shape=jax.ShapeDtypeStruct((ROWS, 2 * D), jnp.float32),
    )(A.T, B.T)


def _sc_gather_side(idx, T):
    mesh = plsc.VectorSubcoreMesh(
        core_axis_name="c", subcore_axis_name="s",
        num_cores=NC, num_subcores=NS)

    @functools.partial(
        pl.kernel,
        out_type=jax.ShapeDtypeStruct((BATCH, 2 * D), jnp.float32),
        mesh=mesh,
        scratch_types=(
            pltpu.VMEM((BPW,), jnp.int32),
            pltpu.VMEM((BPW, 2 * D), jnp.float32),
            pltpu.SemaphoreType.DMA,
        ),
    )
    def k(idx_h, t_h, g_h, idx_v, buf, sem):
        wid = lax.axis_index("s") * NC + lax.axis_index("c")
        base = wid * BPW
        pltpu.sync_copy(idx_h.at[pl.ds(base, BPW)], idx_v)
        cps = [
            pltpu.async_copy(
                t_h.at[idx_v.at[pl.ds(j * CHUNK, CHUNK)]],
                buf.at[pl.ds(j * CHUNK, CHUNK)], sem)
            for j in range(NCHUNK)
        ]
        for cp in cps:
            cp.wait()
        pltpu.sync_copy(buf, g_h.at[pl.ds(base, BPW)])

    return k(idx, T)


def _tc_mlp(gu, gi, W1u, W1i, b1, W2, b2, W3, b3, Wph, Wpm, bp):
    BT = 4096

    def body(gu_r, gi_r, w1u_r, w1i_r, b1_r, w2_r, b2_r, w3_r, b3_r,
             wph_r, wpm_r, bp_r, out_r):
        f32 = jnp.float32
        gu_v = gu_r[...]
        gi_v = gi_r[...]
        h = jnp.maximum(
            jnp.dot(gu_v, w1u_r[...], preferred_element_type=f32)
            + jnp.dot(gi_v, w1i_r[...], preferred_element_type=f32)
            + b1_r[...], 0.0)
        h = jnp.maximum(jnp.dot(h, w2_r[...], preferred_element_type=f32)
                        + b2_r[...], 0.0)
        h = jnp.maximum(jnp.dot(h, w3_r[...], preferred_element_type=f32)
                        + b3_r[...], 0.0)
        pred = (jnp.dot(h, wph_r[...], preferred_element_type=f32)
                + jnp.dot(gu_v * gi_v, wpm_r[...], preferred_element_type=f32)
                + bp_r[0, 0])
        out_r[...] = jax.nn.sigmoid(pred)

    def full(shape):
        return pl.BlockSpec(shape, lambda i: tuple(0 for _ in shape))

    return pl.pallas_call(
        body,
        grid=(BATCH // BT,),
        in_specs=[
            pl.BlockSpec((BT, 2 * D), lambda i: (i, 0)),
            pl.BlockSpec((BT, 2 * D), lambda i: (i, 0)),
            full((2 * D, 64)), full((2 * D, 64)), full((1, 64)),
            full((64, 32)), full((1, 32)),
            full((32, 16)), full((1, 16)),
            full((16, 1)), full((2 * D, 1)), full((1, 1)),
        ],
        out_specs=pl.BlockSpec((BT, 1), lambda i: (i, 0)),
        out_shape=jax.ShapeDtypeStruct((BATCH, 1), jnp.float32),
    )(gu, gi, W1u, W1i, b1, W2, b2, W3, b3, Wph, Wpm, bp)


def kernel(user, item, U_mf, I_mf, U_mlp, I_mlp,
           W1, b1, W2, b2, W3, b3, Wp, bp):
    Tu = _merge_pair(U_mlp, U_mf)
    gu = _sc_gather_side(user, Tu)
    Ti = _merge_pair(I_mlp, I_mf)
    gi = _sc_gather_side(item, Ti)
    zeros = jnp.zeros((D, 64), jnp.float32)
    W1u = jnp.concatenate([W1[:D], zeros], axis=0)
    W1i = jnp.concatenate([zeros, W1[D:]], axis=0)
    Wph = Wp[:16]
    Wpm = jnp.concatenate(
        [jnp.zeros((D, 1), jnp.float32), Wp[16:]], axis=0)
    pred = _tc_mlp(
        gu, gi, W1u, W1i, b1[None, :],
        W2, b2[None, :], W3, b3[None, :],
        Wph, Wpm, bp.reshape(1, 1))
    return pred[:, 0]
